# baseline (speedup 1.0000x reference)
import functools

import jax
import jax.numpy as jnp
from jax import lax
from jax.experimental import pallas as pl
from jax.experimental.pallas import tpu as pltpu
from jax.experimental.pallas import tpu_sc as plsc

D = 32
B = 16384
L = 50
NC = 2
NS = 16
NW = NC * NS

V = 1000001
VB = 512
VP = 1000448
NBLK = VP // VB

BPW = B // NW
CH = 32
ROWS = CH * L
NCHUNK = BPW // CH
G = 80
NG = ROWS // G


def _transpose_body(tt_hbm, out_hbm, in_v, out_v):
    wid = lax.axis_index("s") * NC + lax.axis_index("c")
    nblk_w = (NBLK - wid + NW - 1) // NW

    iota = lax.iota(jnp.int32, 16)

    def blk_body(k, carry):
        blk = wid + k * NW
        v0 = blk * VB
        pltpu.sync_copy(tt_hbm.at[:, pl.ds(v0, VB)], in_v)

        def row_body(v, rcarry):
            vs = jnp.full((16,), v, jnp.int32)
            g0 = plsc.load_gather(in_v, [iota, vs])
            g1 = plsc.load_gather(in_v, [iota + 16, vs])
            out_v[v, pl.ds(0, 16)] = g0
            out_v[v, pl.ds(16, 16)] = g1
            return rcarry

        lax.fori_loop(0, VB, row_body, 0)
        pltpu.sync_copy(out_v, out_hbm.at[pl.ds(v0, VB)])
        return carry

    lax.fori_loop(0, nblk_w, blk_body, 0)


@jax.jit
def _transpose(table_t):
    mesh = plsc.VectorSubcoreMesh(core_axis_name="c", subcore_axis_name="s")
    return pl.kernel(
        _transpose_body,
        out_type=jax.ShapeDtypeStruct((VP, D), jnp.float32),
        mesh=mesh,
        compiler_params=pltpu.CompilerParams(
            use_tc_tiling_on_sc=False, needs_layout_passes=False),
        scratch_types=[
            pltpu.VMEM((D, VB), jnp.float32),
            pltpu.VMEM((VB, D), jnp.float32),
        ],
    )(table_t)


def _embed_bag_body(idx_hbm, len_hbm, table_hbm, out_hbm,
                    idx_v, buf_v, out_v, len_v, sem):
    wid = lax.axis_index("s") * NC + lax.axis_index("c")
    base_b = wid * BPW

    pltpu.sync_copy(len_hbm.at[pl.ds(base_b * 1, BPW)], len_v.at[pl.ds(0, BPW)])

    def chunk_body(c, carry):
        flat_base = pl.multiple_of((base_b + c * CH) * L, 8)
        pltpu.sync_copy(idx_hbm.at[pl.ds(flat_base, ROWS)], idx_v)

        copies = []
        for j in range(NG):
            copies.append(pltpu.async_copy(
                table_hbm.at[idx_v.at[pl.ds(j * G, G)]],
                buf_v.at[pl.ds(j * G, G)],
                sem))
        for cp in copies:
            cp.wait()

        def batch_body(b, bcarry):
            r0 = b * L
            acc0 = buf_v[r0, pl.ds(0, 16)]
            acc1 = buf_v[r0, pl.ds(16, 16)]
            for l in range(1, L):
                acc0 = acc0 + buf_v[r0 + l, pl.ds(0, 16)]
                acc1 = acc1 + buf_v[r0 + l, pl.ds(16, 16)]
            lnv = len_v[pl.ds(c * CH + b, 16)]
            lf = jnp.maximum(lnv[0], 1).astype(jnp.float32)
            out_v[b, pl.ds(0, 16)] = acc0 / lf
            out_v[b, pl.ds(16, 16)] = acc1 / lf
            return bcarry

        lax.fori_loop(0, CH, batch_body, 0)

        out_base = pl.multiple_of(base_b + c * CH, 8)
        pltpu.sync_copy(out_v, out_hbm.at[pl.ds(out_base, CH)])
        return carry

    lax.fori_loop(0, NCHUNK, chunk_body, 0)


@jax.jit
def _embed_bag(idx_flat, len_flat, table_rm):
    mesh = plsc.VectorSubcoreMesh(core_axis_name="c", subcore_axis_name="s")
    return pl.kernel(
        _embed_bag_body,
        out_type=jax.ShapeDtypeStruct((B, D), jnp.float32),
        mesh=mesh,
        compiler_params=pltpu.CompilerParams(use_tc_tiling_on_sc=False),
        scratch_types=[
            pltpu.VMEM((ROWS,), jnp.int32),
            pltpu.VMEM((ROWS, D), jnp.float32),
            pltpu.VMEM((CH, D), jnp.float32),
            pltpu.VMEM((BPW + 16,), jnp.int32),
            pltpu.SemaphoreType.DMA,
        ],
    )(idx_flat, len_flat, table_rm)


def kernel(kw_indices, kw_lengths, embedding_weight):
    idx_flat = kw_indices.reshape(-1).astype(jnp.int32)
    len_flat = kw_lengths.reshape(-1).astype(jnp.int32)
    table_t = jnp.pad(embedding_weight.T, ((0, 0), (0, VP - V)))
    table_rm = _transpose(table_t)
    return _embed_bag(idx_flat, len_flat, table_rm)


# speedup vs baseline: 1.4372x; 1.4372x over previous
import functools

import jax
import jax.numpy as jnp
from jax import lax
from jax.experimental import pallas as pl
from jax.experimental.pallas import tpu as pltpu
from jax.experimental.pallas import tpu_sc as plsc

D = 32
TBLW = 128
V = 1000001
VP = 1000008
B = 16384
L = 50
NC = 2
NS = 16
NW = NC * NS
BPW = B // NW
CH = 8
ROWS = CH * L
NCHUNK = BPW // CH
G = 80
NG = ROWS // G


def _embed_bag_body(idx_hbm, len_hbm, table_hbm, out_hbm,
                    idx_v, buf_v, out_v, len_v, sem):
    wid = lax.axis_index("s") * NC + lax.axis_index("c")
    base_b = wid * BPW

    pltpu.sync_copy(len_hbm.at[pl.ds(base_b * 1, BPW)], len_v.at[pl.ds(0, BPW)])

    def chunk_body(c, carry):
        flat_base = pl.multiple_of((base_b + c * CH) * L, 8)
        pltpu.sync_copy(idx_hbm.at[pl.ds(flat_base, ROWS)], idx_v)

        copies = []
        for j in range(NG):
            copies.append(pltpu.async_copy(
                table_hbm.at[idx_v.at[pl.ds(j * G, G)]],
                buf_v.at[pl.ds(j * G, G)],
                sem))
        for cp in copies:
            cp.wait()

        def batch_body(b, bcarry):
            r0 = b * L
            acc0 = buf_v[r0, pl.ds(0, 16)]
            acc1 = buf_v[r0, pl.ds(16, 16)]
            for l in range(1, L):
                acc0 = acc0 + buf_v[r0 + l, pl.ds(0, 16)]
                acc1 = acc1 + buf_v[r0 + l, pl.ds(16, 16)]
            lnv = len_v[pl.ds(c * CH + b, 16)]
            lf = jnp.maximum(lnv[0], 1).astype(jnp.float32)
            out_v[b, pl.ds(0, 16)] = acc0 / lf
            out_v[b, pl.ds(16, 16)] = acc1 / lf
            return bcarry

        lax.fori_loop(0, CH, batch_body, 0)

        out_base = pl.multiple_of(base_b + c * CH, 8)
        pltpu.sync_copy(out_v, out_hbm.at[pl.ds(out_base, CH)])
        return carry

    lax.fori_loop(0, NCHUNK, chunk_body, 0)


@jax.jit
def _embed_bag(idx_flat, len_flat, table_pad):
    mesh = plsc.VectorSubcoreMesh(core_axis_name="c", subcore_axis_name="s")
    return pl.kernel(
        _embed_bag_body,
        out_type=jax.ShapeDtypeStruct((B, D), jnp.float32),
        mesh=mesh,
        compiler_params=pltpu.CompilerParams(use_tc_tiling_on_sc=False),
        scratch_types=[
            pltpu.VMEM((ROWS,), jnp.int32),
            pltpu.VMEM((ROWS, TBLW), jnp.float32),
            pltpu.VMEM((CH, D), jnp.float32),
            pltpu.VMEM((BPW + 16,), jnp.int32),
            pltpu.SemaphoreType.DMA,
        ],
    )(idx_flat, len_flat, table_pad)


def kernel(kw_indices, kw_lengths, embedding_weight):
    idx_flat = kw_indices.reshape(-1).astype(jnp.int32)
    len_flat = kw_lengths.reshape(-1).astype(jnp.int32)
    table_pad = jnp.pad(embedding_weight, ((0, VP - V), (0, TBLW - D)))
    return _embed_bag(idx_flat, len_flat, table_pad)
